# baseline (speedup 1.0000x reference)
import functools
import math

import jax
import jax.numpy as jnp
from jax import lax
from jax.experimental import pallas as pl
from jax.experimental.pallas import tpu as pltpu
from jax.experimental.pallas import tpu_sc as plsc

N_NODES = 10000
N_EDGES = 320000
HIDDEN = 128
NUM_RBF = 16
CUTOFF = 5.0

NC = 2
NS = 16
NW = NC * NS
E_PER_W = N_EDGES // NW
CHUNK = 40
N_CHUNKS = E_PER_W // CHUNK
NBUF = 3
N_PAD = 10240
ROWS_PER_TILE = N_PAD // NS
ZROWS = 128
LANES = 16

EDGE_BLK = 12800


def _filter_body(eaT_ref, ew_ref, wf1t_ref, bf1_ref, wf2t_ref, bf2_ref, out_ref):
    ea = jnp.transpose(eaT_ref[...])
    t = jnp.tanh(jnp.dot(ea, wf1t_ref[...],
                         preferred_element_type=jnp.float32) + bf1_ref[...])
    wf = jnp.dot(t, wf2t_ref[...], preferred_element_type=jnp.float32) + bf2_ref[...]
    z2 = jnp.square(ew_ref[...] * (math.pi / (2.0 * CUTOFF)))
    p = 1.0 + z2 * (-1.0 / 2.0 + z2 * (1.0 / 24.0 + z2 * (
        -1.0 / 720.0 + z2 * (1.0 / 40320.0 + z2 * (-1.0 / 3628800.0)))))
    c2 = jnp.transpose(p * p)
    out_ref[...] = wf * c2


def _h_body(x_ref, w_ref, out_ref):
    out_ref[...] = jnp.dot(x_ref[...], w_ref[...],
                           preferred_element_type=jnp.float32)


def _tail_body(p_ref, w2_ref, b2_ref, w3_ref, b3_ref, out_ref):
    agg = p_ref[0] + p_ref[1]
    y = jnp.tanh(jnp.dot(agg, w2_ref[...],
                         preferred_element_type=jnp.float32) + b2_ref[...])
    out_ref[...] = jnp.dot(y, w3_ref[...],
                           preferred_element_type=jnp.float32) + b3_ref[...]


def _sc_body(src_hbm, dst_hbm, h_hbm, wf_hbm, out_hbm,
             acc, sidx0, sidx1, sidx2, didx0, didx1, didx2,
             rows0, rows1, rows2, wfb0, wfb1, wfb2, zbuf,
             gsem0, gsem1, gsem2, wsem0, wsem1, wsem2,
             ssem0, ssem1, ssem2, isem0, isem1, isem2):
    c = lax.axis_index("c")
    s = lax.axis_index("s")
    wid = s * NC + c
    sidx = (sidx0, sidx1, sidx2)
    didx = (didx0, didx1, didx2)
    rows = (rows0, rows1, rows2)
    wfb = (wfb0, wfb1, wfb2)
    gsem = (gsem0, gsem1, gsem2)
    wsem = (wsem0, wsem1, wsem2)
    ssem = (ssem0, ssem1, ssem2)
    isem = (isem0, isem1, isem2)

    zero16 = jnp.zeros((LANES,), jnp.float32)

    def _zrow(i, carry):
        for j in range(HIDDEN // LANES):
            zbuf[i, pl.ds(j * LANES, LANES)] = zero16
        return carry

    lax.fori_loop(0, ZROWS, _zrow, 0)
    for k in range(ROWS_PER_TILE // ZROWS):
        pltpu.sync_copy(zbuf, acc.at[pl.ds(s * ROWS_PER_TILE + k * ZROWS, ZROWS)])
    plsc.subcore_barrier()

    def _start_idx(it, b):
        base = pl.multiple_of(wid * E_PER_W + it * CHUNK, CHUNK)
        pltpu.async_copy(src_hbm.at[pl.ds(base, CHUNK)], sidx[b], isem[b])
        pltpu.async_copy(dst_hbm.at[pl.ds(base, CHUNK)], didx[b], isem[b])

    def _wait_idx(it, b):
        base = pl.multiple_of(wid * E_PER_W + it * CHUNK, CHUNK)
        pltpu.make_async_copy(src_hbm.at[pl.ds(base, CHUNK)], sidx[b], isem[b]).wait()
        pltpu.make_async_copy(dst_hbm.at[pl.ds(base, CHUNK)], didx[b], isem[b]).wait()

    def _start_loads(it, b):
        pltpu.async_copy(h_hbm.at[sidx[b]], rows[b], gsem[b])
        base = pl.multiple_of(wid * E_PER_W + it * CHUNK, CHUNK)
        pltpu.async_copy(wf_hbm.at[pl.ds(base, CHUNK)], wfb[b], wsem[b])

    def _wait_loads(it, b):
        pltpu.make_async_copy(h_hbm.at[sidx[b]], rows[b], gsem[b]).wait()
        base = pl.multiple_of(wid * E_PER_W + it * CHUNK, CHUNK)
        pltpu.make_async_copy(wf_hbm.at[pl.ds(base, CHUNK)], wfb[b], wsem[b]).wait()

    def _mul(b):
        def body(e, carry):
            for j in range(HIDDEN // LANES):
                sl = pl.ds(j * LANES, LANES)
                rows[b][e, sl] = rows[b][e, sl] * wfb[b][e, sl]
            return carry
        lax.fori_loop(0, CHUNK, body, 0)

    def _start_scatter(it, b):
        pltpu.async_copy(rows[b], acc.at[didx[b]], ssem[b], add=True)

    def _wait_scatter(b):
        pltpu.make_async_copy(rows[b], acc.at[didx[b]], ssem[b]).wait()

    def _step(it, b, drain, nxt):
        bn = (b + 1) % NBUF
        if drain:
            _wait_scatter(bn)
        if nxt:
            _start_idx(it + 1, bn)
        _wait_loads(it, b)
        _mul(b)
        if nxt:
            _wait_idx(it + 1, bn)
            _start_loads(it + 1, bn)
        _start_scatter(it, b)

    _start_idx(0, 0)
    _wait_idx(0, 0)
    _start_loads(0, 0)
    _step(0, 0, drain=False, nxt=True)
    _step(1, 1, drain=False, nxt=True)

    def _outer(g, carry):
        it0 = NBUF * g + 2
        for d in range(NBUF):
            _step(it0 + d, (2 + d) % NBUF, drain=True, nxt=True)
        return carry

    n_steady = (N_CHUNKS - 4) // NBUF
    lax.fori_loop(0, n_steady, _outer, 0)
    _step(N_CHUNKS - 2, (N_CHUNKS - 2) % NBUF, drain=True, nxt=True)
    _step(N_CHUNKS - 1, (N_CHUNKS - 1) % NBUF, drain=True, nxt=False)

    for it in range(N_CHUNKS - NBUF + 1, N_CHUNKS):
        _wait_scatter(it % NBUF)
    plsc.subcore_barrier()

    for k in range(ROWS_PER_TILE // ZROWS):
        r0 = s * ROWS_PER_TILE + k * ZROWS
        pltpu.sync_copy(acc.at[pl.ds(r0, ZROWS)], zbuf)
        pltpu.sync_copy(zbuf, out_hbm.at[c, pl.ds(r0, ZROWS)])


def kernel(x, edge_index, edge_weight, edge_attr, Wf1, bf1, Wf2, bf2,
           lin1_W, lin2_W, lin2_b, lin_W, lin_b):
    src = edge_index[0].astype(jnp.int32)
    dst = edge_index[1].astype(jnp.int32)
    ew = edge_weight.reshape(1, N_EDGES)

    wfilt = pl.pallas_call(
        _filter_body,
        grid=(N_EDGES // EDGE_BLK,),
        in_specs=[
            pl.BlockSpec((NUM_RBF, EDGE_BLK), lambda i: (0, i)),
            pl.BlockSpec((1, EDGE_BLK), lambda i: (0, i)),
            pl.BlockSpec((NUM_RBF, HIDDEN), lambda i: (0, 0)),
            pl.BlockSpec((1, HIDDEN), lambda i: (0, 0)),
            pl.BlockSpec((HIDDEN, HIDDEN), lambda i: (0, 0)),
            pl.BlockSpec((1, HIDDEN), lambda i: (0, 0)),
        ],
        out_specs=pl.BlockSpec((EDGE_BLK, HIDDEN), lambda i: (i, 0)),
        out_shape=jax.ShapeDtypeStruct((N_EDGES, HIDDEN), jnp.float32),
    )(edge_attr.T, ew, Wf1.T, bf1.reshape(1, HIDDEN), Wf2.T, bf2.reshape(1, HIDDEN))

    h = pl.pallas_call(
        _h_body,
        out_shape=jax.ShapeDtypeStruct((N_NODES, HIDDEN), jnp.float32),
    )(x, lin1_W.T)

    mesh = plsc.VectorSubcoreMesh(core_axis_name="c", subcore_axis_name="s")
    partials = pl.kernel(
        _sc_body,
        out_type=jax.ShapeDtypeStruct((NC, N_PAD, HIDDEN), jnp.float32),
        mesh=mesh,
        scratch_types=[
            pltpu.VMEM_SHARED((N_PAD, HIDDEN), jnp.float32),
            pltpu.VMEM((CHUNK,), jnp.int32),
            pltpu.VMEM((CHUNK,), jnp.int32),
            pltpu.VMEM((CHUNK,), jnp.int32),
            pltpu.VMEM((CHUNK,), jnp.int32),
            pltpu.VMEM((CHUNK,), jnp.int32),
            pltpu.VMEM((CHUNK,), jnp.int32),
            pltpu.VMEM((CHUNK, HIDDEN), jnp.float32),
            pltpu.VMEM((CHUNK, HIDDEN), jnp.float32),
            pltpu.VMEM((CHUNK, HIDDEN), jnp.float32),
            pltpu.VMEM((CHUNK, HIDDEN), jnp.float32),
            pltpu.VMEM((CHUNK, HIDDEN), jnp.float32),
            pltpu.VMEM((CHUNK, HIDDEN), jnp.float32),
            pltpu.VMEM((ZROWS, HIDDEN), jnp.float32),
        ] + [pltpu.SemaphoreType.DMA] * 12,
    )(src, dst, h, wfilt)

    out = pl.pallas_call(
        _tail_body,
        out_shape=jax.ShapeDtypeStruct((N_PAD, HIDDEN), jnp.float32),
    )(partials, lin2_W.T, lin2_b.reshape(1, HIDDEN), lin_W.T,
      lin_b.reshape(1, HIDDEN))
    return out[:N_NODES]


# speedup vs baseline: 1.0712x; 1.0712x over previous
import functools
import math

import jax
import jax.numpy as jnp
from jax import lax
from jax.experimental import pallas as pl
from jax.experimental.pallas import tpu as pltpu
from jax.experimental.pallas import tpu_sc as plsc

N_NODES = 10000
N_EDGES = 320000
HIDDEN = 128
NUM_RBF = 16
CUTOFF = 5.0

NC = 2
NS = 16
NW = NC * NS
N_HALVES = 2
E_HALF = N_EDGES // N_HALVES
E_PER_W = E_HALF // NW
CHUNK = 40
N_CHUNKS = E_PER_W // CHUNK
NBUF = 3
N_PAD = 10240
ROWS_PER_TILE = N_PAD // NS
ZROWS = 128
LANES = 16

EDGE_BLK = 16000
FILT_BLOCKS = E_HALF // EDGE_BLK


def _filter_body(eaT_ref, ew_ref, wf1t_ref, bf1_ref, wf2t_ref, bf2_ref, out_ref):
    ea = jnp.transpose(eaT_ref[...])
    t = jnp.tanh(jnp.dot(ea, wf1t_ref[...],
                         preferred_element_type=jnp.float32) + bf1_ref[...])
    wf = jnp.dot(t, wf2t_ref[...], preferred_element_type=jnp.float32) + bf2_ref[...]
    z2 = jnp.square(ew_ref[...] * (math.pi / (2.0 * CUTOFF)))
    p = 1.0 + z2 * (-1.0 / 2.0 + z2 * (1.0 / 24.0 + z2 * (
        -1.0 / 720.0 + z2 * (1.0 / 40320.0 + z2 * (-1.0 / 3628800.0)))))
    c2 = jnp.transpose(p * p)
    out_ref[...] = wf * c2


def _h_body(x_ref, w_ref, out_ref):
    out_ref[...] = jnp.dot(x_ref[...], w_ref[...],
                           preferred_element_type=jnp.float32)


def _tail_body(pa_ref, pb_ref, w2_ref, b2_ref, w3_ref, b3_ref, out_ref):
    agg = (pa_ref[0] + pa_ref[1]) + (pb_ref[0] + pb_ref[1])
    y = jnp.tanh(jnp.dot(agg, w2_ref[...],
                         preferred_element_type=jnp.float32) + b2_ref[...])
    out_ref[...] = jnp.dot(y, w3_ref[...],
                           preferred_element_type=jnp.float32) + b3_ref[...]


def _make_sc_body(half_base):
    def _sc_body(src_hbm, dst_hbm, h_hbm, wf_hbm, out_hbm,
                 acc, sidx0, sidx1, sidx2, didx0, didx1, didx2,
                 rows0, rows1, rows2, wfb0, wfb1, wfb2, zbuf,
                 gsem0, gsem1, gsem2, wsem0, wsem1, wsem2,
                 ssem0, ssem1, ssem2, isem0, isem1, isem2):
        c = lax.axis_index("c")
        s = lax.axis_index("s")
        wid = s * NC + c
        sidx = (sidx0, sidx1, sidx2)
        didx = (didx0, didx1, didx2)
        rows = (rows0, rows1, rows2)
        wfb = (wfb0, wfb1, wfb2)
        gsem = (gsem0, gsem1, gsem2)
        wsem = (wsem0, wsem1, wsem2)
        ssem = (ssem0, ssem1, ssem2)
        isem = (isem0, isem1, isem2)

        zero16 = jnp.zeros((LANES,), jnp.float32)

        def _zrow(i, carry):
            for j in range(HIDDEN // LANES):
                zbuf[i, pl.ds(j * LANES, LANES)] = zero16
            return carry

        lax.fori_loop(0, ZROWS, _zrow, 0)
        for k in range(ROWS_PER_TILE // ZROWS):
            pltpu.sync_copy(zbuf,
                            acc.at[pl.ds(s * ROWS_PER_TILE + k * ZROWS, ZROWS)])
        plsc.subcore_barrier()

        def _start_idx(it, b):
            gb = pl.multiple_of(half_base + wid * E_PER_W + it * CHUNK, CHUNK)
            pltpu.async_copy(src_hbm.at[pl.ds(gb, CHUNK)], sidx[b], isem[b])
            pltpu.async_copy(dst_hbm.at[pl.ds(gb, CHUNK)], didx[b], isem[b])

        def _wait_idx(it, b):
            gb = pl.multiple_of(half_base + wid * E_PER_W + it * CHUNK, CHUNK)
            pltpu.make_async_copy(src_hbm.at[pl.ds(gb, CHUNK)], sidx[b],
                                  isem[b]).wait()
            pltpu.make_async_copy(dst_hbm.at[pl.ds(gb, CHUNK)], didx[b],
                                  isem[b]).wait()

        def _start_loads(it, b):
            pltpu.async_copy(h_hbm.at[sidx[b]], rows[b], gsem[b])
            base = pl.multiple_of(wid * E_PER_W + it * CHUNK, CHUNK)
            pltpu.async_copy(wf_hbm.at[pl.ds(base, CHUNK)], wfb[b], wsem[b])

        def _wait_loads(it, b):
            pltpu.make_async_copy(h_hbm.at[sidx[b]], rows[b], gsem[b]).wait()
            base = pl.multiple_of(wid * E_PER_W + it * CHUNK, CHUNK)
            pltpu.make_async_copy(wf_hbm.at[pl.ds(base, CHUNK)], wfb[b],
                                  wsem[b]).wait()

        def _mul(b):
            def body(e, carry):
                for j in range(HIDDEN // LANES):
                    sl = pl.ds(j * LANES, LANES)
                    rows[b][e, sl] = rows[b][e, sl] * wfb[b][e, sl]
                return carry
            lax.fori_loop(0, CHUNK, body, 0)

        def _start_scatter(it, b):
            pltpu.async_copy(rows[b], acc.at[didx[b]], ssem[b], add=True)

        def _wait_scatter(b):
            pltpu.make_async_copy(rows[b], acc.at[didx[b]], ssem[b]).wait()

        def _step(it, b, drain, nxt):
            bn = (b + 1) % NBUF
            if drain:
                _wait_scatter(bn)
            if nxt:
                _start_idx(it + 1, bn)
            _wait_loads(it, b)
            _mul(b)
            if nxt:
                _wait_idx(it + 1, bn)
                _start_loads(it + 1, bn)
            _start_scatter(it, b)

        _start_idx(0, 0)
        _wait_idx(0, 0)
        _start_loads(0, 0)
        _step(0, 0, drain=False, nxt=True)
        _step(1, 1, drain=False, nxt=True)

        n_steady = (N_CHUNKS - 4) // NBUF

        def _outer(g, carry):
            it0 = NBUF * g + 2
            for d in range(NBUF):
                _step(it0 + d, (2 + d) % NBUF, drain=True, nxt=True)
            return carry

        lax.fori_loop(0, n_steady, _outer, 0)
        for it in range(2 + NBUF * n_steady, N_CHUNKS):
            _step(it, it % NBUF, drain=True, nxt=(it + 1 < N_CHUNKS))

        for it in range(N_CHUNKS - NBUF + 1, N_CHUNKS):
            _wait_scatter(it % NBUF)
        plsc.subcore_barrier()

        for k in range(ROWS_PER_TILE // ZROWS):
            r0 = s * ROWS_PER_TILE + k * ZROWS
            pltpu.sync_copy(acc.at[pl.ds(r0, ZROWS)], zbuf)
            pltpu.sync_copy(zbuf, out_hbm.at[c, pl.ds(r0, ZROWS)])

    return _sc_body


_SC_SCRATCH = [
    pltpu.VMEM_SHARED((N_PAD, HIDDEN), jnp.float32),
    pltpu.VMEM((CHUNK,), jnp.int32),
    pltpu.VMEM((CHUNK,), jnp.int32),
    pltpu.VMEM((CHUNK,), jnp.int32),
    pltpu.VMEM((CHUNK,), jnp.int32),
    pltpu.VMEM((CHUNK,), jnp.int32),
    pltpu.VMEM((CHUNK,), jnp.int32),
    pltpu.VMEM((CHUNK, HIDDEN), jnp.float32),
    pltpu.VMEM((CHUNK, HIDDEN), jnp.float32),
    pltpu.VMEM((CHUNK, HIDDEN), jnp.float32),
    pltpu.VMEM((CHUNK, HIDDEN), jnp.float32),
    pltpu.VMEM((CHUNK, HIDDEN), jnp.float32),
    pltpu.VMEM((CHUNK, HIDDEN), jnp.float32),
    pltpu.VMEM((ZROWS, HIDDEN), jnp.float32),
] + [pltpu.SemaphoreType.DMA] * 12


def kernel(x, edge_index, edge_weight, edge_attr, Wf1, bf1, Wf2, bf2,
           lin1_W, lin2_W, lin2_b, lin_W, lin_b):
    src = edge_index[0].astype(jnp.int32)
    dst = edge_index[1].astype(jnp.int32)
    eaT = edge_attr.T
    ew = edge_weight.reshape(1, N_EDGES)
    filt_args = (Wf1.T, bf1.reshape(1, HIDDEN), Wf2.T, bf2.reshape(1, HIDDEN))

    def filter_half(half):
        return pl.pallas_call(
            _filter_body,
            grid=(FILT_BLOCKS,),
            in_specs=[
                pl.BlockSpec((NUM_RBF, EDGE_BLK),
                             lambda i, h=half: (0, i + h * FILT_BLOCKS)),
                pl.BlockSpec((1, EDGE_BLK),
                             lambda i, h=half: (0, i + h * FILT_BLOCKS)),
                pl.BlockSpec((NUM_RBF, HIDDEN), lambda i: (0, 0)),
                pl.BlockSpec((1, HIDDEN), lambda i: (0, 0)),
                pl.BlockSpec((HIDDEN, HIDDEN), lambda i: (0, 0)),
                pl.BlockSpec((1, HIDDEN), lambda i: (0, 0)),
            ],
            out_specs=pl.BlockSpec((EDGE_BLK, HIDDEN), lambda i: (i, 0)),
            out_shape=jax.ShapeDtypeStruct((E_HALF, HIDDEN), jnp.float32),
        )(eaT, ew, *filt_args)

    h = pl.pallas_call(
        _h_body,
        out_shape=jax.ShapeDtypeStruct((N_NODES, HIDDEN), jnp.float32),
    )(x, lin1_W.T)

    mesh = plsc.VectorSubcoreMesh(core_axis_name="c", subcore_axis_name="s")

    def sc_half(half, wf):
        return pl.kernel(
            _make_sc_body(half * E_HALF),
            out_type=jax.ShapeDtypeStruct((NC, N_PAD, HIDDEN), jnp.float32),
            mesh=mesh,
            scratch_types=_SC_SCRATCH,
        )(src, dst, h, wf)

    wf0 = filter_half(0)
    pa = sc_half(0, wf0)
    wf1 = filter_half(1)
    pb = sc_half(1, wf1)

    out = pl.pallas_call(
        _tail_body,
        out_shape=jax.ShapeDtypeStruct((N_PAD, HIDDEN), jnp.float32),
    )(pa, pb, lin2_W.T, lin2_b.reshape(1, HIDDEN), lin_W.T,
      lin_b.reshape(1, HIDDEN))
    return out[:N_NODES]


# speedup vs baseline: 1.5058x; 1.4057x over previous
import functools
import math

import jax
import jax.numpy as jnp
from jax import lax
from jax.experimental import pallas as pl
from jax.experimental.pallas import tpu as pltpu
from jax.experimental.pallas import tpu_sc as plsc

N_NODES = 10000
N_EDGES = 320000
HIDDEN = 128
NUM_RBF = 16
CUTOFF = 5.0

NC = 2
NS = 16
NW = NC * NS
N_HALVES = 2
E_HALF = N_EDGES // N_HALVES
E_PER_W = E_HALF // NW
CHUNK = 40
N_CHUNKS = E_PER_W // CHUNK
NBUF = 3
IBUF = 4
UNROLL = 12
N_PAD = 10240
ROWS_PER_TILE = N_PAD // NS
ZROWS = 128
LANES = 16

EDGE_BLK = 16000
FILT_BLOCKS = E_HALF // EDGE_BLK


def _filter_body(eaT_ref, ew_ref, wf1t_ref, bf1_ref, wf2t_ref, bf2_ref, out_ref):
    ea = jnp.transpose(eaT_ref[...])
    t = jnp.tanh(jnp.dot(ea, wf1t_ref[...],
                         preferred_element_type=jnp.float32) + bf1_ref[...])
    wf = jnp.dot(t, wf2t_ref[...], preferred_element_type=jnp.float32) + bf2_ref[...]
    z2 = jnp.square(ew_ref[...] * (math.pi / (2.0 * CUTOFF)))
    p = 1.0 + z2 * (-1.0 / 2.0 + z2 * (1.0 / 24.0 + z2 * (
        -1.0 / 720.0 + z2 * (1.0 / 40320.0 + z2 * (-1.0 / 3628800.0)))))
    c2 = jnp.transpose(p * p)
    out_ref[...] = wf * c2


def _h_body(x_ref, w_ref, out_ref):
    out_ref[...] = jnp.dot(x_ref[...], w_ref[...],
                           preferred_element_type=jnp.float32)


def _tail_body(pa_ref, pb_ref, w2_ref, b2_ref, w3_ref, b3_ref, out_ref):
    agg = (pa_ref[0] + pa_ref[1]) + (pb_ref[0] + pb_ref[1])
    y = jnp.tanh(jnp.dot(agg, w2_ref[...],
                         preferred_element_type=jnp.float32) + b2_ref[...])
    out_ref[...] = jnp.dot(y, w3_ref[...],
                           preferred_element_type=jnp.float32) + b3_ref[...]


def _make_sc_body(half_base):
    def _sc_body(src_hbm, dst_hbm, h_hbm, wf_hbm, out_hbm,
                 acc, sidx0, sidx1, sidx2, sidx3, didx0, didx1, didx2, didx3,
                 rows0, rows1, rows2, wfb0, wfb1, wfb2, zbuf,
                 gsem0, gsem1, gsem2, wsem0, wsem1, wsem2,
                 ssem0, ssem1, ssem2, isem0, isem1, isem2, isem3):
        c = lax.axis_index("c")
        s = lax.axis_index("s")
        wid = s * NC + c
        sidx = (sidx0, sidx1, sidx2, sidx3)
        didx = (didx0, didx1, didx2, didx3)
        rows = (rows0, rows1, rows2)
        wfb = (wfb0, wfb1, wfb2)
        gsem = (gsem0, gsem1, gsem2)
        wsem = (wsem0, wsem1, wsem2)
        ssem = (ssem0, ssem1, ssem2)
        isem = (isem0, isem1, isem2, isem3)

        zero16 = jnp.zeros((LANES,), jnp.float32)

        def _zrow(i, carry):
            for j in range(HIDDEN // LANES):
                zbuf[i, pl.ds(j * LANES, LANES)] = zero16
            return carry

        lax.fori_loop(0, ZROWS, _zrow, 0)
        for k in range(ROWS_PER_TILE // ZROWS):
            pltpu.sync_copy(zbuf,
                            acc.at[pl.ds(s * ROWS_PER_TILE + k * ZROWS, ZROWS)])
        plsc.subcore_barrier()

        def _start_idx(it, b):
            gb = pl.multiple_of(half_base + wid * E_PER_W + it * CHUNK, CHUNK)
            pltpu.async_copy(src_hbm.at[pl.ds(gb, CHUNK)], sidx[b], isem[b])
            pltpu.async_copy(dst_hbm.at[pl.ds(gb, CHUNK)], didx[b], isem[b])

        def _wait_idx(it, b):
            gb = pl.multiple_of(half_base + wid * E_PER_W + it * CHUNK, CHUNK)
            pltpu.make_async_copy(src_hbm.at[pl.ds(gb, CHUNK)], sidx[b],
                                  isem[b]).wait()
            pltpu.make_async_copy(dst_hbm.at[pl.ds(gb, CHUNK)], didx[b],
                                  isem[b]).wait()

        def _start_loads(it, b, sl):
            pltpu.async_copy(h_hbm.at[sidx[sl]], rows[b], gsem[b])
            base = pl.multiple_of(wid * E_PER_W + it * CHUNK, CHUNK)
            pltpu.async_copy(wf_hbm.at[pl.ds(base, CHUNK)], wfb[b], wsem[b])

        def _wait_loads(it, b, sl):
            pltpu.make_async_copy(h_hbm.at[sidx[sl]], rows[b], gsem[b]).wait()
            base = pl.multiple_of(wid * E_PER_W + it * CHUNK, CHUNK)
            pltpu.make_async_copy(wf_hbm.at[pl.ds(base, CHUNK)], wfb[b],
                                  wsem[b]).wait()

        def _mul(b):
            def body(e, carry):
                for j in range(HIDDEN // LANES):
                    sl = pl.ds(j * LANES, LANES)
                    rows[b][e, sl] = rows[b][e, sl] * wfb[b][e, sl]
                return carry
            lax.fori_loop(0, CHUNK, body, 0)

        def _start_scatter(it, b, sl):
            pltpu.async_copy(rows[b], acc.at[didx[sl]], ssem[b], add=True)

        def _wait_scatter(b):
            pltpu.make_async_copy(rows[b], acc.at[didx[0]], ssem[b]).wait()

        def _step(it, b3, b4, drain, idx2, nxt):
            bn = (b3 + 1) % NBUF
            if drain:
                _wait_scatter(bn)
            if idx2:
                _start_idx(it + 2, (b4 + 2) % IBUF)
            if nxt:
                _wait_idx(it + 1, (b4 + 1) % IBUF)
                _start_loads(it + 1, bn, (b4 + 1) % IBUF)
            _wait_loads(it, b3, b4)
            _mul(b3)
            _start_scatter(it, b3, b4)

        _start_idx(0, 0)
        _wait_idx(0, 0)
        _start_idx(1, 1)
        _start_loads(0, 0, 0)
        _step(0, 0, 0, drain=False, idx2=True, nxt=True)
        _step(1, 1, 1, drain=False, idx2=True, nxt=True)

        n_steady = (N_CHUNKS - 5) // UNROLL

        def _outer(g, carry):
            it0 = UNROLL * g + 2
            for d in range(UNROLL):
                _step(it0 + d, (2 + d) % NBUF, (2 + d) % IBUF,
                      drain=True, idx2=True, nxt=True)
            return carry

        lax.fori_loop(0, n_steady, _outer, 0)
        for it in range(2 + UNROLL * n_steady, N_CHUNKS):
            _step(it, it % NBUF, it % IBUF, drain=True,
                  idx2=(it + 2 < N_CHUNKS), nxt=(it + 1 < N_CHUNKS))

        for it in range(N_CHUNKS - NBUF + 1, N_CHUNKS):
            _wait_scatter(it % NBUF)
        plsc.subcore_barrier()

        for k in range(ROWS_PER_TILE // ZROWS):
            r0 = s * ROWS_PER_TILE + k * ZROWS
            pltpu.sync_copy(acc.at[pl.ds(r0, ZROWS)], zbuf)
            pltpu.sync_copy(zbuf, out_hbm.at[c, pl.ds(r0, ZROWS)])

    return _sc_body


_SC_SCRATCH = (
    [pltpu.VMEM_SHARED((N_PAD, HIDDEN), jnp.float32)]
    + [pltpu.VMEM((CHUNK,), jnp.int32)] * (2 * IBUF)
    + [pltpu.VMEM((CHUNK, HIDDEN), jnp.float32)] * (2 * NBUF)
    + [pltpu.VMEM((ZROWS, HIDDEN), jnp.float32)]
    + [pltpu.SemaphoreType.DMA] * (3 * NBUF + IBUF)
)


def kernel(x, edge_index, edge_weight, edge_attr, Wf1, bf1, Wf2, bf2,
           lin1_W, lin2_W, lin2_b, lin_W, lin_b):
    src = edge_index[0].astype(jnp.int32)
    dst = edge_index[1].astype(jnp.int32)
    eaT = edge_attr.T
    ew = edge_weight.reshape(1, N_EDGES)
    filt_args = (Wf1.T, bf1.reshape(1, HIDDEN), Wf2.T, bf2.reshape(1, HIDDEN))

    def filter_half(half):
        return pl.pallas_call(
            _filter_body,
            grid=(FILT_BLOCKS,),
            in_specs=[
                pl.BlockSpec((NUM_RBF, EDGE_BLK),
                             lambda i, h=half: (0, i + h * FILT_BLOCKS)),
                pl.BlockSpec((1, EDGE_BLK),
                             lambda i, h=half: (0, i + h * FILT_BLOCKS)),
                pl.BlockSpec((NUM_RBF, HIDDEN), lambda i: (0, 0)),
                pl.BlockSpec((1, HIDDEN), lambda i: (0, 0)),
                pl.BlockSpec((HIDDEN, HIDDEN), lambda i: (0, 0)),
                pl.BlockSpec((1, HIDDEN), lambda i: (0, 0)),
            ],
            out_specs=pl.BlockSpec((EDGE_BLK, HIDDEN), lambda i: (i, 0)),
            out_shape=jax.ShapeDtypeStruct((E_HALF, HIDDEN), jnp.float32),
        )(eaT, ew, *filt_args)

    h = pl.pallas_call(
        _h_body,
        out_shape=jax.ShapeDtypeStruct((N_NODES, HIDDEN), jnp.float32),
    )(x, lin1_W.T)

    mesh = plsc.VectorSubcoreMesh(core_axis_name="c", subcore_axis_name="s")

    def sc_half(half, wf):
        return pl.kernel(
            _make_sc_body(half * E_HALF),
            out_type=jax.ShapeDtypeStruct((NC, N_PAD, HIDDEN), jnp.float32),
            mesh=mesh,
            scratch_types=_SC_SCRATCH,
        )(src, dst, h, wf)

    wf0 = filter_half(0)
    pa = sc_half(0, wf0)
    wf1 = filter_half(1)
    pb = sc_half(1, wf1)

    out = pl.pallas_call(
        _tail_body,
        out_shape=jax.ShapeDtypeStruct((N_PAD, HIDDEN), jnp.float32),
    )(pa, pb, lin2_W.T, lin2_b.reshape(1, HIDDEN), lin_W.T,
      lin_b.reshape(1, HIDDEN))
    return out[:N_NODES]
